# baseline (speedup 1.0000x reference)
import functools

import jax
import jax.numpy as jnp
from jax import lax
from jax.experimental import pallas as pl
from jax.experimental.pallas import tpu as pltpu
from jax.experimental.pallas import tpu_sc as plsc

N = 100_000
E = 6_400_000
LANES = 16
CHUNK = 1600
VPC = CHUNK // LANES
NC = 2
NS = 16
NW = NC * NS
CPW = E // CHUNK // NW
UNROLL = 4
STEPS = 30
TAIL = CPW - STEPS * UNROLL
SLICE = 6256
LAST_SLICE = N - (NS - 1) * SLICE
PIECE = CHUNK

_mesh = plsc.VectorSubcoreMesh(
    core_axis_name="c", subcore_axis_name="s", num_cores=NC, num_subcores=NS
)


@functools.partial(
    pl.kernel,
    out_type=jax.ShapeDtypeStruct((NC * N,), jnp.float32),
    mesh=_mesh,
    scratch_types=[
        pltpu.VMEM((N,), jnp.float32),
        pltpu.VMEM((2 * CHUNK,), jnp.int32),
        pltpu.VMEM((2 * CHUNK,), jnp.int32),
        pltpu.VMEM((2 * CHUNK,), jnp.float32),
        pltpu.VMEM((2 * CHUNK,), jnp.float32),
        pltpu.VMEM((CHUNK,), jnp.int32),
        pltpu.VMEM((CHUNK,), jnp.int32),
        pltpu.VMEM((CHUNK,), jnp.int32),
        pltpu.VMEM((CHUNK,), jnp.int32),
        pltpu.VMEM((CHUNK,), jnp.float32),
        pltpu.VMEM((CHUNK,), jnp.float32),
        pltpu.VMEM_SHARED((N,), jnp.float32),
        pltpu.SemaphoreType.DMA,
        pltpu.SemaphoreType.DMA,
        pltpu.SemaphoreType.DMA,
    ],
    compiler_params=pltpu.CompilerParams(needs_layout_passes=False),
)
def _spmv_sc(ed_hbm, row_hbm, col_hbm, b_hbm, out_hbm,
             b_v, col_p0, col_p1, ed_p0, ed_p1,
             row_v0, row_v1, row_v2, row_v3,
             prod_v0, prod_v1,
             acc, sem_in, sem_row, sem_sc):
    col_p = (col_p0, col_p1)
    ed_p = (ed_p0, ed_p1)
    row_v = (row_v0, row_v1, row_v2, row_v3)
    prod_v = (prod_v0, prod_v1)
    c = lax.axis_index("c")
    s = lax.axis_index("s")
    wid = s * NC + c

    @pl.when(s < NS - 1)
    def _():
        for p0 in range(0, SLICE, PIECE):
            w = min(PIECE, SLICE - p0)
            pltpu.sync_copy(b_hbm.at[pl.ds(s * SLICE + p0, w)],
                            prod_v0.at[pl.ds(0, w)])
            pltpu.sync_copy(prod_v0.at[pl.ds(0, w)],
                            acc.at[pl.ds(s * SLICE + p0, w)])

    @pl.when(s == NS - 1)
    def _():
        for p0 in range(0, LAST_SLICE, PIECE):
            w = min(PIECE, LAST_SLICE - p0)
            pltpu.sync_copy(b_hbm.at[pl.ds((NS - 1) * SLICE + p0, w)],
                            prod_v0.at[pl.ds(0, w)])
            pltpu.sync_copy(prod_v0.at[pl.ds(0, w)],
                            acc.at[pl.ds((NS - 1) * SLICE + p0, w)])

    plsc.subcore_barrier()
    pltpu.sync_copy(acc, b_v)
    plsc.subcore_barrier()

    def zero_body(k, carry):
        prod_v0[pl.ds(k * LANES, LANES)] = jnp.zeros((LANES,), jnp.float32)
        return carry

    lax.fori_loop(0, PIECE // LANES, zero_body, 0)

    @pl.when(s < NS - 1)
    def _():
        for p0 in range(0, SLICE, PIECE):
            w = min(PIECE, SLICE - p0)
            pltpu.sync_copy(prod_v0.at[pl.ds(0, w)],
                            acc.at[pl.ds(s * SLICE + p0, w)])

    @pl.when(s == NS - 1)
    def _():
        for p0 in range(0, LAST_SLICE, PIECE):
            w = min(PIECE, LAST_SLICE - p0)
            pltpu.sync_copy(prod_v0.at[pl.ds(0, w)],
                            acc.at[pl.ds((NS - 1) * SLICE + p0, w)])

    def fire_pair(p, b2):
        e0 = wid * (CPW * CHUNK) + p * (2 * CHUNK)
        sl = pl.ds(e0, 2 * CHUNK)
        pltpu.async_copy(col_hbm.at[sl], col_p[b2], sem_in)
        pltpu.async_copy(ed_hbm.at[sl], ed_p[b2], sem_in)

    def fire_last_singles():
        e0 = wid * (CPW * CHUNK) + (CPW - 1) * CHUNK
        sl = pl.ds(e0, CHUNK)
        pltpu.async_copy(col_hbm.at[sl], col_p[0].at[pl.ds(0, CHUNK)], sem_in)
        pltpu.async_copy(ed_hbm.at[sl], ed_p[0].at[pl.ds(0, CHUNK)], sem_in)

    def fire_row(i, b4):
        e0 = wid * (CPW * CHUNK) + i * CHUNK
        pltpu.async_copy(row_hbm.at[pl.ds(e0, CHUNK)], row_v[b4], sem_row)

    def wait_pair():
        pltpu.make_async_copy(ed_hbm.at[pl.ds(0, 4 * CHUNK)],
                              b_v.at[pl.ds(0, 4 * CHUNK)], sem_in).wait()

    def wait_singles():
        pltpu.make_async_copy(ed_hbm.at[pl.ds(0, 2 * CHUNK)],
                              b_v.at[pl.ds(0, 2 * CHUNK)], sem_in).wait()

    def wait_row():
        pltpu.make_async_copy(ed_hbm.at[pl.ds(0, CHUNK)],
                              prod_v1, sem_row).wait()

    def drain_scatter():
        pltpu.make_async_copy(ed_hbm.at[pl.ds(0, CHUNK)],
                              prod_v0, sem_sc).wait()

    def chunk_body(i, q, drain, fire_mode, fire2, last=False):
        if q % 2 == 0:
            if last:
                wait_singles()
            else:
                wait_pair()
        wait_row()
        if drain:
            drain_scatter()
        if fire_mode == 'pair':
            fire_pair((i + 2) // 2, ((q // 2) + 1) % 2)
        elif fire_mode == 'singles':
            fire_last_singles()
        if fire2:
            fire_row(i + 2, (q + 2) % 4)
        bp, off, b4, b2 = (q // 2) % 2, (q % 2) * CHUNK, q % 4, q % 2
        if last:
            bp, off = 0, 0
        cp, ep = col_p[bp], ed_p[bp]

        @plsc.parallel_loop(0, VPC, unroll=8)
        def _(k):
            sl = pl.ds(off + k * LANES, LANES)
            bvals = plsc.load_gather(b_v, [cp[sl]])
            prod_v[b2][pl.ds(k * LANES, LANES)] = ep[sl] * bvals

        pltpu.async_copy(prod_v[b2], acc.at[row_v[b4]], sem_sc, add=True)

    fire_pair(0, 0)
    fire_pair(1, 1)
    fire_row(0, 0)
    fire_row(1, 1)
    plsc.subcore_barrier()

    for q in range(UNROLL):
        chunk_body(q, q, drain=q >= 2,
                   fire_mode='pair' if q in (2,) else None, fire2=True)

    def step_body(p, carry):
        base = p * UNROLL
        for q in range(UNROLL):
            chunk_body(base + q, q, drain=True,
                       fire_mode='pair' if q % 2 == 0 else None, fire2=True)
        return carry

    lax.fori_loop(1, STEPS, step_body, 0)

    chunk_body(120, 0, drain=True, fire_mode='pair', fire2=True)
    chunk_body(121, 1, drain=True, fire_mode=None, fire2=True)
    chunk_body(122, 2, drain=True, fire_mode='singles', fire2=True)
    chunk_body(123, 3, drain=True, fire_mode=None, fire2=False)
    chunk_body(124, 0, drain=True, fire_mode=None, fire2=False, last=True)
    drain_scatter()
    drain_scatter()

    plsc.subcore_barrier()

    @pl.when(s < NS - 1)
    def _():
        for p0 in range(0, SLICE, PIECE):
            w = min(PIECE, SLICE - p0)
            pltpu.sync_copy(acc.at[pl.ds(s * SLICE + p0, w)],
                            prod_v0.at[pl.ds(0, w)])
            pltpu.sync_copy(prod_v0.at[pl.ds(0, w)],
                            out_hbm.at[pl.ds(c * N + s * SLICE + p0, w)])

    @pl.when(s == NS - 1)
    def _():
        for p0 in range(0, LAST_SLICE, PIECE):
            w = min(PIECE, LAST_SLICE - p0)
            pltpu.sync_copy(acc.at[pl.ds((NS - 1) * SLICE + p0, w)],
                            prod_v0.at[pl.ds(0, w)])
            pltpu.sync_copy(
                prod_v0.at[pl.ds(0, w)],
                out_hbm.at[pl.ds(c * N + (NS - 1) * SLICE + p0, w)])


def _combine_body(p_ref, o_ref):
    o_ref[...] = p_ref[0:1, :] + p_ref[1:2, :]


def kernel(edata, row, col, B):
    partial = _spmv_sc(edata, row, col, B).reshape(NC, N)
    out = pl.pallas_call(
        _combine_body,
        out_shape=jax.ShapeDtypeStruct((1, N), jnp.float32),
    )(partial)
    return out.reshape(N)


# speedup vs baseline: 1.0126x; 1.0126x over previous
import functools

import jax
import jax.numpy as jnp
from jax import lax
from jax.experimental import pallas as pl
from jax.experimental.pallas import tpu as pltpu
from jax.experimental.pallas import tpu_sc as plsc

N = 100_000
E = 6_400_000
LANES = 16
CHUNK = 1600
VPC = CHUNK // LANES
NC = 2
NS = 16
NW = NC * NS
CPW = E // CHUNK // NW
UNROLL = 4
STEPS = 30
TAIL = CPW - STEPS * UNROLL
SLICE = 6256
LAST_SLICE = N - (NS - 1) * SLICE
PIECE = CHUNK

_mesh = plsc.VectorSubcoreMesh(
    core_axis_name="c", subcore_axis_name="s", num_cores=NC, num_subcores=NS
)


@functools.partial(
    pl.kernel,
    out_type=jax.ShapeDtypeStruct((NC * N,), jnp.float32),
    mesh=_mesh,
    scratch_types=[
        pltpu.VMEM((N,), jnp.float32),
        pltpu.VMEM((CHUNK,), jnp.int32),
        pltpu.VMEM((CHUNK,), jnp.int32),
        pltpu.VMEM((CHUNK,), jnp.int32),
        pltpu.VMEM((CHUNK,), jnp.int32),
        pltpu.VMEM((CHUNK,), jnp.float32),
        pltpu.VMEM((CHUNK,), jnp.float32),
        pltpu.VMEM((CHUNK,), jnp.float32),
        pltpu.VMEM((CHUNK,), jnp.float32),
        pltpu.VMEM((CHUNK,), jnp.int32),
        pltpu.VMEM((CHUNK,), jnp.int32),
        pltpu.VMEM((CHUNK,), jnp.int32),
        pltpu.VMEM((CHUNK,), jnp.int32),
        pltpu.VMEM((CHUNK,), jnp.float32),
        pltpu.VMEM((CHUNK,), jnp.float32),
        pltpu.VMEM_SHARED((N,), jnp.float32),
        pltpu.SemaphoreType.DMA,
        pltpu.SemaphoreType.DMA,
        pltpu.SemaphoreType.DMA,
    ],
    compiler_params=pltpu.CompilerParams(needs_layout_passes=False),
)
def _spmv_sc(ed_hbm, row_hbm, col_hbm, b_hbm, out_hbm,
             b_v, col_v0, col_v1, col_v2, col_v3,
             ed_v0, ed_v1, ed_v2, ed_v3,
             row_v0, row_v1, row_v2, row_v3,
             prod_v0, prod_v1,
             acc, sem_in, sem_row, sem_sc):
    col_v = (col_v0, col_v1, col_v2, col_v3)
    ed_v = (ed_v0, ed_v1, ed_v2, ed_v3)
    row_v = (row_v0, row_v1, row_v2, row_v3)
    prod_v = (prod_v0, prod_v1)
    c = lax.axis_index("c")
    s = lax.axis_index("s")
    wid = s * NC + c

    @pl.when(s < NS - 1)
    def _():
        for p0 in range(0, SLICE, PIECE):
            w = min(PIECE, SLICE - p0)
            pltpu.sync_copy(b_hbm.at[pl.ds(s * SLICE + p0, w)],
                            prod_v0.at[pl.ds(0, w)])
            pltpu.sync_copy(prod_v0.at[pl.ds(0, w)],
                            acc.at[pl.ds(s * SLICE + p0, w)])

    @pl.when(s == NS - 1)
    def _():
        for p0 in range(0, LAST_SLICE, PIECE):
            w = min(PIECE, LAST_SLICE - p0)
            pltpu.sync_copy(b_hbm.at[pl.ds((NS - 1) * SLICE + p0, w)],
                            prod_v0.at[pl.ds(0, w)])
            pltpu.sync_copy(prod_v0.at[pl.ds(0, w)],
                            acc.at[pl.ds((NS - 1) * SLICE + p0, w)])

    plsc.subcore_barrier()
    pltpu.sync_copy(acc, b_v)
    plsc.subcore_barrier()

    def zero_body(k, carry):
        prod_v0[pl.ds(k * LANES, LANES)] = jnp.zeros((LANES,), jnp.float32)
        return carry

    lax.fori_loop(0, PIECE // LANES, zero_body, 0)

    @pl.when(s < NS - 1)
    def _():
        for p0 in range(0, SLICE, PIECE):
            w = min(PIECE, SLICE - p0)
            pltpu.sync_copy(prod_v0.at[pl.ds(0, w)],
                            acc.at[pl.ds(s * SLICE + p0, w)])

    @pl.when(s == NS - 1)
    def _():
        for p0 in range(0, LAST_SLICE, PIECE):
            w = min(PIECE, LAST_SLICE - p0)
            pltpu.sync_copy(prod_v0.at[pl.ds(0, w)],
                            acc.at[pl.ds((NS - 1) * SLICE + p0, w)])

    def fire_col_ed(i, b4):
        e0 = (wid * CPW + i) * CHUNK
        sl = pl.ds(e0, CHUNK)
        pltpu.async_copy(col_hbm.at[sl], col_v[b4], sem_in)
        pltpu.async_copy(ed_hbm.at[sl], ed_v[b4], sem_in)

    def fire_row(i, b4):
        e0 = (wid * CPW + i) * CHUNK
        pltpu.async_copy(row_hbm.at[pl.ds(e0, CHUNK)], row_v[b4], sem_row)

    def wait_loads():
        pltpu.make_async_copy(ed_hbm.at[pl.ds(0, 2 * CHUNK)],
                              b_v.at[pl.ds(0, 2 * CHUNK)], sem_in).wait()
        pltpu.make_async_copy(ed_hbm.at[pl.ds(0, CHUNK)],
                              prod_v1, sem_row).wait()

    def drain_scatter():
        pltpu.make_async_copy(ed_hbm.at[pl.ds(0, CHUNK)],
                              prod_v0, sem_sc).wait()

    def chunk_body(i, q, drain, fire3, fire2):
        wait_loads()
        if drain:
            drain_scatter()
        if fire3:
            fire_col_ed(i + 3, (q + 3) % 4)
        if fire2:
            fire_row(i + 2, (q + 2) % 4)
        b4, b2 = q % 4, q % 2

        @plsc.parallel_loop(0, VPC, unroll=8)
        def _(k):
            sl = pl.ds(k * LANES, LANES)
            bvals = plsc.load_gather(b_v, [col_v[b4][sl]])
            prod_v[b2][sl] = ed_v[b4][sl] * bvals

        pltpu.async_copy(prod_v[b2], acc.at[row_v[b4]], sem_sc, add=True)

    fire_col_ed(0, 0)
    fire_col_ed(1, 1)
    fire_col_ed(2, 2)
    fire_row(0, 0)
    fire_row(1, 1)
    plsc.subcore_barrier()

    for q in range(UNROLL):
        chunk_body(q, q, drain=q >= 2, fire3=True, fire2=True)

    def step_body(p, carry):
        base = p * UNROLL
        for q in range(UNROLL):
            chunk_body(base + q, q, drain=True, fire3=True, fire2=True)
        return carry

    lax.fori_loop(1, STEPS, step_body, 0)

    for q in range(TAIL):
        chunk_body(STEPS * UNROLL + q, q, drain=True,
                   fire3=q < TAIL - 3, fire2=q < TAIL - 2)
    drain_scatter()
    drain_scatter()

    plsc.subcore_barrier()

    @pl.when(s < NS - 1)
    def _():
        for p0 in range(0, SLICE, PIECE):
            w = min(PIECE, SLICE - p0)
            pltpu.sync_copy(acc.at[pl.ds(s * SLICE + p0, w)],
                            prod_v0.at[pl.ds(0, w)])
            pltpu.sync_copy(prod_v0.at[pl.ds(0, w)],
                            out_hbm.at[pl.ds(c * N + s * SLICE + p0, w)])

    @pl.when(s == NS - 1)
    def _():
        for p0 in range(0, LAST_SLICE, PIECE):
            w = min(PIECE, LAST_SLICE - p0)
            pltpu.sync_copy(acc.at[pl.ds((NS - 1) * SLICE + p0, w)],
                            prod_v0.at[pl.ds(0, w)])
            pltpu.sync_copy(
                prod_v0.at[pl.ds(0, w)],
                out_hbm.at[pl.ds(c * N + (NS - 1) * SLICE + p0, w)])


def _combine_body(p_ref, o_ref):
    o_ref[...] = p_ref[0:1, :] + p_ref[1:2, :]


def kernel(edata, row, col, B):
    partial = _spmv_sc(edata, row, col, B).reshape(NC, N)
    out = pl.pallas_call(
        _combine_body,
        out_shape=jax.ShapeDtypeStruct((1, N), jnp.float32),
    )(partial)
    return out.reshape(N)
